# baseline (speedup 1.0000x reference)
import functools

import jax
import jax.numpy as jnp
from jax.experimental import pallas as pl
from jax.experimental.pallas import tpu as pltpu
from jax.experimental.pallas import tpu_sc as plsc

N = 10000
D = 128
HID = 128
OUT = 256
RATIO = 0.5
R = 0.2
MAX_NB = 32
M = int(N * RATIO)

SUB = 8
LANES = -(-N // (SUB * 128)) * 128
TOTAL = SUB * LANES


def _fps_body(posr_ref, xg_ref, yg_ref, zg_ref, idx_ref, cen_ref):
    sub_i = jax.lax.broadcasted_iota(jnp.int32, (SUB, LANES), 0)
    lane_i = jax.lax.broadcasted_iota(jnp.int32, (SUB, LANES), 1)
    iota = sub_i * LANES + lane_i
    valid = iota < N
    big = jnp.int32(N)
    dd0 = jnp.where(valid, jnp.inf, -jnp.inf).astype(jnp.float32)

    idx_ref[pl.ds(0, 1), :] = jnp.zeros((1, 1), jnp.int32)
    cen_ref[pl.ds(0, 1), :] = posr_ref[pl.ds(0, 1), :]

    def body(i, carry):
        dd, last = carry
        rowp = posr_ref[pl.ds(last, 1), :]
        bx = jnp.broadcast_to(rowp[0:1, 0:1], (SUB, LANES))
        by = jnp.broadcast_to(rowp[0:1, 1:2], (SUB, LANES))
        bz = jnp.broadcast_to(rowp[0:1, 2:3], (SUB, LANES))
        dx = xg_ref[:, :] - bx
        dy = yg_ref[:, :] - by
        dz = zg_ref[:, :] - bz
        d = dx * dx + dy * dy + dz * dz
        dd = jnp.minimum(dd, d)
        mx = jnp.max(dd)
        nxt = jnp.min(jnp.where(dd == mx, iota, big)).astype(jnp.int32)
        idx_ref[pl.ds(i, 1), :] = jnp.full((1, 1), 0, jnp.int32) + nxt
        cen_ref[pl.ds(i, 1), :] = posr_ref[pl.ds(nxt, 1), :]
        return (dd, nxt)

    jax.lax.fori_loop(1, M, body, (dd0, jnp.int32(0)))


def _fps_pallas(pos):
    pad = jnp.zeros((TOTAL - N,), jnp.float32)
    xg = jnp.concatenate([pos[:, 0], pad]).reshape(SUB, LANES)
    yg = jnp.concatenate([pos[:, 1], pad]).reshape(SUB, LANES)
    zg = jnp.concatenate([pos[:, 2], pad]).reshape(SUB, LANES)
    idx2, cen = pl.pallas_call(
        _fps_body,
        out_shape=(
            jax.ShapeDtypeStruct((M, 1), jnp.int32),
            jax.ShapeDtypeStruct((M, 3), jnp.float32),
        ),
    )(pos, xg, yg, zg)
    return idx2[:, 0], cen


QBITS = 17
JBITS = 14
JMASK = (1 << JBITS) - 1
KSCALE = float((2 ** QBITS - 1) / (R * R))
SENT = 2 ** 31 - 1
PPAD = 10240
MPAD = 5120
CBLK = 256
PBLK = 2560


def _keys_body(cen_ref, posT_ref, pn2_ref, keys_ref):
    c = cen_ref[:, :]
    pT = posT_ref[:, :]
    pn2 = pn2_ref[:, :]
    cn2 = jnp.sum(c * c, axis=1, keepdims=True)
    dot = jnp.dot(c, pT, preferred_element_type=jnp.float32)
    d2 = cn2 + pn2 - 2.0 * dot
    q = jnp.maximum(jnp.int32(0), (d2 * KSCALE).astype(jnp.int32))
    j = jax.lax.broadcasted_iota(jnp.int32, (CBLK, PBLK), 1)
    base = pl.program_id(1) * PBLK
    key = jnp.where(d2 <= R * R, (q << JBITS) | (j + base), jnp.int32(SENT))
    keys_ref[:, :] = key


def _keys_pallas(centers, pos):
    cenp = jnp.concatenate(
        [centers, jnp.full((MPAD - M, 3), 100.0, jnp.float32)], axis=0)
    posT = jnp.concatenate(
        [pos.T, jnp.full((3, PPAD - N), 50.0, jnp.float32)], axis=1)
    pn2 = jnp.sum(posT * posT, axis=0, keepdims=True)
    keys = pl.pallas_call(
        _keys_body,
        grid=(MPAD // CBLK, PPAD // PBLK),
        in_specs=[
            pl.BlockSpec((CBLK, 3), lambda i, j: (i, 0)),
            pl.BlockSpec((3, PBLK), lambda i, j: (0, j)),
            pl.BlockSpec((1, PBLK), lambda i, j: (0, j)),
        ],
        out_specs=pl.BlockSpec((CBLK, PBLK), lambda i, j: (i, j)),
        out_shape=jax.ShapeDtypeStruct((MPAD, PPAD), jnp.int32),
    )(cenp, posT, pn2)
    return keys


def _select_topk(keys):
    skey = -jax.lax.top_k(-keys, MAX_NB)[0]
    valid = skey != SENT
    nbr = jnp.where(valid, skey & JMASK, 0)
    return nbr, valid


TBLC = 256
YBLK = 1000


def _table_body(x_ref, pos_ref, wm_ref, bm_ref, tab_ref):
    xb = x_ref[:, :]
    y = jnp.dot(xb, wm_ref[:, :], preferred_element_type=jnp.float32) + bm_ref[:, :]
    tab_ref[:, 0:D] = y
    tab_ref[:, D:D + 3] = pos_ref[:, :]
    tab_ref[:, D + 3:TBLC] = jnp.zeros((YBLK, TBLC - D - 3), jnp.float32)


def _table_pallas(x, pos, W_msg, b_msg):
    tab = pl.pallas_call(
        _table_body,
        grid=(N // YBLK,),
        in_specs=[
            pl.BlockSpec((YBLK, D), lambda i: (i, 0)),
            pl.BlockSpec((YBLK, 3), lambda i: (i, 0)),
            pl.BlockSpec((D, HID), lambda i: (0, 0)),
            pl.BlockSpec((1, HID), lambda i: (0, 0)),
        ],
        out_specs=pl.BlockSpec((YBLK, TBLC), lambda i: (i, 0)),
        out_shape=jax.ShapeDtypeStruct((N, TBLC), jnp.float32),
    )(x, pos, W_msg[:D], b_msg[None, :])
    return tab


SC_NW = 32
EDGES = MAX_NB * MPAD
E_PER_W = EDGES // SC_NW
ECHUNK = 128
NECH = E_PER_W // ECHUNK
C_PER_W = MPAD // SC_NW


def _sc_gather_body(tab_hbm, x_hbm, bat_hbm, colt_hbm, idx_hbm,
                    ge_hbm, gx_hbm, gb_hbm,
                    idx_v, rows_v, xrows_v, brows_v, sem):
    c = jax.lax.axis_index("c")
    s = jax.lax.axis_index("s")
    wid = s * 2 + c
    ebase = wid * E_PER_W

    def echunk(t, carry):
        off = ebase + t * ECHUNK
        pltpu.sync_copy(colt_hbm.at[pl.ds(off, ECHUNK)], idx_v)
        pltpu.async_copy(tab_hbm.at[idx_v], rows_v, sem).wait()
        pltpu.sync_copy(rows_v, ge_hbm.at[pl.ds(off, ECHUNK)])
        return carry

    jax.lax.fori_loop(0, NECH, echunk, 0)

    cbase = wid * C_PER_W
    pltpu.sync_copy(idx_hbm.at[pl.ds(cbase, 128)], idx_v)
    pltpu.async_copy(x_hbm.at[idx_v], xrows_v, sem).wait()
    pltpu.sync_copy(xrows_v, gx_hbm.at[pl.ds(cbase, 128)])
    pltpu.async_copy(bat_hbm.at[idx_v], brows_v, sem).wait()
    pltpu.sync_copy(brows_v, gb_hbm.at[pl.ds(cbase, 128)])

    idx_v2 = idx_v.at[pl.ds(0, 32)]
    pltpu.sync_copy(idx_hbm.at[pl.ds(cbase + 128, 32)], idx_v2)
    xrows_v2 = xrows_v.at[pl.ds(0, 32)]
    pltpu.async_copy(x_hbm.at[idx_v2], xrows_v2, sem).wait()
    pltpu.sync_copy(xrows_v2, gx_hbm.at[pl.ds(cbase + 128, 32)])
    brows_v2 = brows_v.at[pl.ds(0, 32)]
    pltpu.async_copy(bat_hbm.at[idx_v2], brows_v2, sem).wait()
    pltpu.sync_copy(brows_v2, gb_hbm.at[pl.ds(cbase + 128, 32)])


def _sc_gather(tab, x, batpad, col_t, idx_pad):
    mesh = plsc.VectorSubcoreMesh(core_axis_name="c", subcore_axis_name="s")
    fn = functools.partial(
        pl.kernel,
        mesh=mesh,
        out_type=(
            jax.ShapeDtypeStruct((EDGES, TBLC), jnp.float32),
            jax.ShapeDtypeStruct((MPAD, D), jnp.float32),
            jax.ShapeDtypeStruct((MPAD, 128), jnp.int32),
        ),
        scratch_types=[
            pltpu.VMEM((ECHUNK,), jnp.int32),
            pltpu.VMEM((ECHUNK, TBLC), jnp.float32),
            pltpu.VMEM((128, D), jnp.float32),
            pltpu.VMEM((128, 128), jnp.int32),
            pltpu.SemaphoreType.DMA,
        ],
    )(_sc_gather_body)
    return fn(tab, x, batpad, col_t, idx_pad)


EBLK = 128


def _edge_body(ge_ref, vm_ref, cen_ref, gx_ref, wl_ref, wp_ref, bp_ref,
               bu_ref, wua_ref, wub_ref, xo_ref, po_ref):
    wlast = wl_ref[:, :]
    cen = cen_ref[:, :]
    cnt = jnp.sum(vm_ref[:, :], axis=1, keepdims=True)

    def kstep(k, carry):
        acc_x, acc_p = carry
        yk = ge_ref[k, :, 0:D]
        pj = ge_ref[k, :, D:D + 3]
        diff = pj - cen
        d2e = jnp.sum(diff * diff, axis=1, keepdims=True)
        dist = jnp.sqrt(d2e + 1e-12)
        e = jax.nn.relu(yk + dist * wlast)
        w3 = jnp.dot(e, wp_ref[:, :], preferred_element_type=jnp.float32) + bp_ref[:, :]
        vm = jnp.clip(cnt - k.astype(jnp.float32), 0.0, 1.0)
        acc_x = acc_x + e * vm
        acc_p = acc_p + diff * w3 * vm
        return (acc_x, acc_p)

    acc_x0 = jnp.zeros((EBLK, HID), jnp.float32)
    acc_p0 = jnp.zeros((EBLK, 3), jnp.float32)
    acc_x, acc_p = jax.lax.fori_loop(0, MAX_NB, kstep, (acc_x0, acc_p0))
    xo = (jnp.dot(gx_ref[:, :], wua_ref[:, :], preferred_element_type=jnp.float32)
          + jnp.dot(acc_x, wub_ref[:, :], preferred_element_type=jnp.float32)
          + bu_ref[:, :])
    xo_ref[:, :] = jax.nn.relu(xo)
    po_ref[:, :] = cen + acc_p / jnp.maximum(cnt, 1.0)


def _edge_pallas(ge, vmask, centers_pad, gx, W_msg, W_pos, b_pos, b_upd, W_upd):
    ge3 = ge.reshape(MAX_NB, MPAD, TBLC)
    xo, po = pl.pallas_call(
        _edge_body,
        grid=(MPAD // EBLK,),
        in_specs=[
            pl.BlockSpec((MAX_NB, EBLK, TBLC), lambda i: (0, i, 0)),
            pl.BlockSpec((EBLK, MAX_NB), lambda i: (i, 0)),
            pl.BlockSpec((EBLK, 3), lambda i: (i, 0)),
            pl.BlockSpec((EBLK, D), lambda i: (i, 0)),
            pl.BlockSpec((1, HID), lambda i: (0, 0)),
            pl.BlockSpec((HID, 3), lambda i: (0, 0)),
            pl.BlockSpec((1, 3), lambda i: (0, 0)),
            pl.BlockSpec((1, OUT), lambda i: (0, 0)),
            pl.BlockSpec((D, OUT), lambda i: (0, 0)),
            pl.BlockSpec((HID, OUT), lambda i: (0, 0)),
        ],
        out_specs=(
            pl.BlockSpec((EBLK, OUT), lambda i: (i, 0)),
            pl.BlockSpec((EBLK, 3), lambda i: (i, 0)),
        ),
        out_shape=(
            jax.ShapeDtypeStruct((MPAD, OUT), jnp.float32),
            jax.ShapeDtypeStruct((MPAD, 3), jnp.float32),
        ),
    )(ge3, vmask, centers_pad, gx, W_msg[D][None, :], W_pos, b_pos[None, :],
      b_upd[None, :], W_upd[:D], W_upd[D:])
    return xo[:M], po[:M]


def kernel(x, pos, W_msg, b_msg, W_pos, b_pos, W_upd, b_upd, batch):
    idx, centers = _fps_pallas(pos)
    keys = _keys_pallas(centers, pos)
    nbr, valid = _select_topk(keys)
    vmask = valid.astype(jnp.float32)
    col_t = nbr.T.reshape(-1)
    tab = _table_pallas(x, pos, W_msg, b_msg)
    batpad = jnp.broadcast_to(batch[:, None], (N, 128)).astype(jnp.int32)
    idx_pad = jnp.concatenate([idx, jnp.zeros((MPAD - M,), jnp.int32)])
    ge, gx, gb = _sc_gather(tab, x, batpad, col_t, idx_pad)
    centers_pad = jnp.concatenate(
        [centers, jnp.zeros((MPAD - M, 3), jnp.float32)], axis=0)
    x_out, pos_out = _edge_pallas(ge, vmask, centers_pad, gx, W_msg, W_pos,
                                  b_pos, b_upd, W_upd)
    batch_out = gb[:M, 0]
    return (x_out, pos_out, batch_out)


# speedup vs baseline: 1.0098x; 1.0098x over previous
import functools

import jax
import jax.numpy as jnp
from jax.experimental import pallas as pl
from jax.experimental.pallas import tpu as pltpu
from jax.experimental.pallas import tpu_sc as plsc

N = 10000
D = 128
HID = 128
OUT = 256
RATIO = 0.5
R = 0.2
MAX_NB = 32
M = int(N * RATIO)

SUB = 8
LANES = -(-N // (SUB * 128)) * 128
TOTAL = SUB * LANES


def _fps_body(posr_ref, xg_ref, yg_ref, zg_ref, idx_ref, cen_ref):
    sub_i = jax.lax.broadcasted_iota(jnp.int32, (SUB, LANES), 0)
    lane_i = jax.lax.broadcasted_iota(jnp.int32, (SUB, LANES), 1)
    iota = sub_i * LANES + lane_i
    valid = iota < N
    big = jnp.int32(N)
    dd0 = jnp.where(valid, jnp.inf, -jnp.inf).astype(jnp.float32)

    idx_ref[pl.ds(0, 1), :] = jnp.zeros((1, 1), jnp.int32)
    cen_ref[pl.ds(0, 1), :] = posr_ref[pl.ds(0, 1), :]

    def body(i, carry):
        dd, last = carry
        rowp = posr_ref[pl.ds(last, 1), :]
        bx = jnp.broadcast_to(rowp[0:1, 0:1], (SUB, LANES))
        by = jnp.broadcast_to(rowp[0:1, 1:2], (SUB, LANES))
        bz = jnp.broadcast_to(rowp[0:1, 2:3], (SUB, LANES))
        dx = xg_ref[:, :] - bx
        dy = yg_ref[:, :] - by
        dz = zg_ref[:, :] - bz
        d = dx * dx + dy * dy + dz * dz
        dd = jnp.minimum(dd, d)
        mx = jnp.max(dd)
        nxt = jnp.min(jnp.where(dd == mx, iota, big)).astype(jnp.int32)
        idx_ref[pl.ds(i, 1), :] = jnp.full((1, 1), 0, jnp.int32) + nxt
        cen_ref[pl.ds(i, 1), :] = posr_ref[pl.ds(nxt, 1), :]
        return (dd, nxt)

    jax.lax.fori_loop(1, M, body, (dd0, jnp.int32(0)))


def _fps_pallas(pos):
    pad = jnp.zeros((TOTAL - N,), jnp.float32)
    xg = jnp.concatenate([pos[:, 0], pad]).reshape(SUB, LANES)
    yg = jnp.concatenate([pos[:, 1], pad]).reshape(SUB, LANES)
    zg = jnp.concatenate([pos[:, 2], pad]).reshape(SUB, LANES)
    idx2, cen = pl.pallas_call(
        _fps_body,
        out_shape=(
            jax.ShapeDtypeStruct((M, 1), jnp.int32),
            jax.ShapeDtypeStruct((M, 3), jnp.float32),
        ),
    )(pos, xg, yg, zg)
    return idx2[:, 0], cen


QBITS = 17
JBITS = 14
JMASK = (1 << JBITS) - 1
KSCALE = float((2 ** QBITS - 1) / (R * R))
SENT = 2 ** 31 - 1
PPAD = 10240
MPAD = 5120
CBLK = 256
PBLK = 2560


def _keys_body(cen_ref, posT_ref, pn2_ref, keys_ref):
    c = cen_ref[:, :]
    pT = posT_ref[:, :]
    pn2 = pn2_ref[:, :]
    cn2 = jnp.sum(c * c, axis=1, keepdims=True)
    dot = jnp.dot(c, pT, preferred_element_type=jnp.float32)
    d2 = cn2 + pn2 - 2.0 * dot
    q = jnp.maximum(jnp.int32(0), (d2 * KSCALE).astype(jnp.int32))
    j = jax.lax.broadcasted_iota(jnp.int32, (CBLK, PBLK), 1)
    base = pl.program_id(1) * PBLK
    key = jnp.where(d2 <= R * R, (q << JBITS) | (j + base), jnp.int32(SENT))
    keys_ref[:, :] = key


def _keys_pallas(centers, pos):
    cenp = jnp.concatenate(
        [centers, jnp.full((MPAD - M, 3), 100.0, jnp.float32)], axis=0)
    posT = jnp.concatenate(
        [pos.T, jnp.full((3, PPAD - N), 50.0, jnp.float32)], axis=1)
    pn2 = jnp.sum(posT * posT, axis=0, keepdims=True)
    keys = pl.pallas_call(
        _keys_body,
        grid=(MPAD // CBLK, PPAD // PBLK),
        in_specs=[
            pl.BlockSpec((CBLK, 3), lambda i, j: (i, 0)),
            pl.BlockSpec((3, PBLK), lambda i, j: (0, j)),
            pl.BlockSpec((1, PBLK), lambda i, j: (0, j)),
        ],
        out_specs=pl.BlockSpec((CBLK, PBLK), lambda i, j: (i, j)),
        out_shape=jax.ShapeDtypeStruct((MPAD, PPAD), jnp.int32),
    )(cenp, posT, pn2)
    return keys


def _select_topk(keys):
    skey = -jax.lax.top_k(-keys, MAX_NB)[0]
    valid = skey != SENT
    nbr = jnp.where(valid, skey & JMASK, 0)
    return nbr, valid


TBLC = 256
YBLK = 1000


def _table_body(x_ref, pos_ref, wm_ref, bm_ref, tab_ref):
    xb = x_ref[:, :]
    y = jnp.dot(xb, wm_ref[:, :], preferred_element_type=jnp.float32) + bm_ref[:, :]
    tab_ref[:, 0:D] = y
    tab_ref[:, D:D + 3] = pos_ref[:, :]
    tab_ref[:, D + 3:TBLC] = jnp.zeros((YBLK, TBLC - D - 3), jnp.float32)


def _table_pallas(x, pos, W_msg, b_msg):
    tab = pl.pallas_call(
        _table_body,
        grid=(N // YBLK,),
        in_specs=[
            pl.BlockSpec((YBLK, D), lambda i: (i, 0)),
            pl.BlockSpec((YBLK, 3), lambda i: (i, 0)),
            pl.BlockSpec((D, HID), lambda i: (0, 0)),
            pl.BlockSpec((1, HID), lambda i: (0, 0)),
        ],
        out_specs=pl.BlockSpec((YBLK, TBLC), lambda i: (i, 0)),
        out_shape=jax.ShapeDtypeStruct((N, TBLC), jnp.float32),
    )(x, pos, W_msg[:D], b_msg[None, :])
    return tab


SC_NW = 32
EDGES = MAX_NB * MPAD
E_PER_W = EDGES // SC_NW
ECHUNK = 128
NECH = E_PER_W // ECHUNK
C_PER_W = MPAD // SC_NW


def _sc_gather_body(tab_hbm, x_hbm, bat_hbm, colt_hbm, idx_hbm,
                    ge_hbm, gx_hbm, gb_hbm,
                    idx_v, idx_v2, rows_v, rows_v2, xrows_v, brows_v,
                    sem, sem2):
    c = jax.lax.axis_index("c")
    s = jax.lax.axis_index("s")
    wid = s * 2 + c
    ebase = wid * E_PER_W

    ibufs = (idx_v, idx_v2)
    rbufs = (rows_v, rows_v2)
    sems = (sem, sem2)

    pltpu.sync_copy(colt_hbm.at[pl.ds(ebase, ECHUNK)], idx_v)
    pltpu.async_copy(tab_hbm.at[idx_v], rows_v, sem)

    def echunk(t, carry):
        cur = jax.lax.rem(t, 2)
        for b in range(2):

            @pl.when(cur == b)
            def _():
                nxt_i, nxt_r, nxt_s = ibufs[1 - b], rbufs[1 - b], sems[1 - b]

                @pl.when(t + 1 < NECH)
                def _():
                    noff = ebase + (t + 1) * ECHUNK
                    pltpu.sync_copy(colt_hbm.at[pl.ds(noff, ECHUNK)], nxt_i)
                    pltpu.async_copy(tab_hbm.at[nxt_i], nxt_r, nxt_s)

                pltpu.make_async_copy(tab_hbm.at[ibufs[b]], rbufs[b], sems[b]).wait()
                off = ebase + t * ECHUNK
                pltpu.sync_copy(rbufs[b], ge_hbm.at[pl.ds(off, ECHUNK)])

        return carry

    jax.lax.fori_loop(0, NECH, echunk, 0)

    cbase = wid * C_PER_W
    pltpu.sync_copy(idx_hbm.at[pl.ds(cbase, 128)], idx_v)
    pltpu.async_copy(x_hbm.at[idx_v], xrows_v, sem).wait()
    pltpu.sync_copy(xrows_v, gx_hbm.at[pl.ds(cbase, 128)])
    pltpu.async_copy(bat_hbm.at[idx_v], brows_v, sem).wait()
    pltpu.sync_copy(brows_v, gb_hbm.at[pl.ds(cbase, 128)])

    idx_v2 = idx_v.at[pl.ds(0, 32)]
    pltpu.sync_copy(idx_hbm.at[pl.ds(cbase + 128, 32)], idx_v2)
    xrows_v2 = xrows_v.at[pl.ds(0, 32)]
    pltpu.async_copy(x_hbm.at[idx_v2], xrows_v2, sem).wait()
    pltpu.sync_copy(xrows_v2, gx_hbm.at[pl.ds(cbase + 128, 32)])
    brows_v2 = brows_v.at[pl.ds(0, 32)]
    pltpu.async_copy(bat_hbm.at[idx_v2], brows_v2, sem).wait()
    pltpu.sync_copy(brows_v2, gb_hbm.at[pl.ds(cbase + 128, 32)])


def _sc_gather(tab, x, batpad, col_t, idx_pad):
    mesh = plsc.VectorSubcoreMesh(core_axis_name="c", subcore_axis_name="s")
    fn = functools.partial(
        pl.kernel,
        mesh=mesh,
        out_type=(
            jax.ShapeDtypeStruct((EDGES, TBLC), jnp.float32),
            jax.ShapeDtypeStruct((MPAD, D), jnp.float32),
            jax.ShapeDtypeStruct((MPAD, 128), jnp.int32),
        ),
        scratch_types=[
            pltpu.VMEM((ECHUNK,), jnp.int32),
            pltpu.VMEM((ECHUNK,), jnp.int32),
            pltpu.VMEM((ECHUNK, TBLC), jnp.float32),
            pltpu.VMEM((ECHUNK, TBLC), jnp.float32),
            pltpu.VMEM((128, D), jnp.float32),
            pltpu.VMEM((128, 128), jnp.int32),
            pltpu.SemaphoreType.DMA,
            pltpu.SemaphoreType.DMA,
        ],
    )(_sc_gather_body)
    return fn(tab, x, batpad, col_t, idx_pad)


EBLK = 128


def _edge_body(ge_ref, vm_ref, cen_ref, gx_ref, wl_ref, wp_ref, bp_ref,
               bu_ref, wua_ref, wub_ref, xo_ref, po_ref):
    wlast = wl_ref[:, :]
    cen = cen_ref[:, :]
    cnt = jnp.sum(vm_ref[:, :], axis=1, keepdims=True)

    def kstep(k, carry):
        acc_x, acc_p = carry
        yk = ge_ref[k, :, 0:D]
        pj = ge_ref[k, :, D:D + 3]
        diff = pj - cen
        d2e = jnp.sum(diff * diff, axis=1, keepdims=True)
        dist = jnp.sqrt(d2e + 1e-12)
        e = jax.nn.relu(yk + dist * wlast)
        w3 = jnp.dot(e, wp_ref[:, :], preferred_element_type=jnp.float32) + bp_ref[:, :]
        vm = jnp.clip(cnt - k.astype(jnp.float32), 0.0, 1.0)
        acc_x = acc_x + e * vm
        acc_p = acc_p + diff * w3 * vm
        return (acc_x, acc_p)

    acc_x0 = jnp.zeros((EBLK, HID), jnp.float32)
    acc_p0 = jnp.zeros((EBLK, 3), jnp.float32)
    acc_x, acc_p = jax.lax.fori_loop(0, MAX_NB, kstep, (acc_x0, acc_p0))
    xo = (jnp.dot(gx_ref[:, :], wua_ref[:, :], preferred_element_type=jnp.float32)
          + jnp.dot(acc_x, wub_ref[:, :], preferred_element_type=jnp.float32)
          + bu_ref[:, :])
    xo_ref[:, :] = jax.nn.relu(xo)
    po_ref[:, :] = cen + acc_p / jnp.maximum(cnt, 1.0)


def _edge_pallas(ge, vmask, centers_pad, gx, W_msg, W_pos, b_pos, b_upd, W_upd):
    ge3 = ge.reshape(MAX_NB, MPAD, TBLC)
    xo, po = pl.pallas_call(
        _edge_body,
        grid=(MPAD // EBLK,),
        in_specs=[
            pl.BlockSpec((MAX_NB, EBLK, TBLC), lambda i: (0, i, 0)),
            pl.BlockSpec((EBLK, MAX_NB), lambda i: (i, 0)),
            pl.BlockSpec((EBLK, 3), lambda i: (i, 0)),
            pl.BlockSpec((EBLK, D), lambda i: (i, 0)),
            pl.BlockSpec((1, HID), lambda i: (0, 0)),
            pl.BlockSpec((HID, 3), lambda i: (0, 0)),
            pl.BlockSpec((1, 3), lambda i: (0, 0)),
            pl.BlockSpec((1, OUT), lambda i: (0, 0)),
            pl.BlockSpec((D, OUT), lambda i: (0, 0)),
            pl.BlockSpec((HID, OUT), lambda i: (0, 0)),
        ],
        out_specs=(
            pl.BlockSpec((EBLK, OUT), lambda i: (i, 0)),
            pl.BlockSpec((EBLK, 3), lambda i: (i, 0)),
        ),
        out_shape=(
            jax.ShapeDtypeStruct((MPAD, OUT), jnp.float32),
            jax.ShapeDtypeStruct((MPAD, 3), jnp.float32),
        ),
    )(ge3, vmask, centers_pad, gx, W_msg[D][None, :], W_pos, b_pos[None, :],
      b_upd[None, :], W_upd[:D], W_upd[D:])
    return xo[:M], po[:M]


def kernel(x, pos, W_msg, b_msg, W_pos, b_pos, W_upd, b_upd, batch):
    idx, centers = _fps_pallas(pos)
    keys = _keys_pallas(centers, pos)
    nbr, valid = _select_topk(keys)
    vmask = valid.astype(jnp.float32)
    col_t = nbr.T.reshape(-1)
    tab = _table_pallas(x, pos, W_msg, b_msg)
    batpad = jnp.broadcast_to(batch[:, None], (N, 128)).astype(jnp.int32)
    idx_pad = jnp.concatenate([idx, jnp.zeros((MPAD - M,), jnp.int32)])
    ge, gx, gb = _sc_gather(tab, x, batpad, col_t, idx_pad)
    centers_pad = jnp.concatenate(
        [centers, jnp.zeros((MPAD - M, 3), jnp.float32)], axis=0)
    x_out, pos_out = _edge_pallas(ge, vmask, centers_pad, gx, W_msg, W_pos,
                                  b_pos, b_upd, W_upd)
    batch_out = gb[:M, 0]
    return (x_out, pos_out, batch_out)
